# baseline (speedup 1.0000x reference)
import jax
import jax.numpy as jnp
from jax.experimental import pallas as pl
from jax.experimental.pallas import tpu as pltpu


def _moe_kernel(
    x_hbm, logits_hbm, w13_hbm, w2_hbm, out_ref,
    x_ref, logits_ref, coeff_ref, g_buf, u_buf, w2_buf, sems, io_sem
):
    num_experts, inter, hidden = w2_hbm.shape[0], w2_hbm.shape[2], w2_hbm.shape[1]

    def start_g(e, slot):
        pltpu.make_async_copy(
            w13_hbm.at[e, pl.ds(0, inter)], g_buf.at[slot], sems.at[0, slot]
        ).start()

    def start_u(e, slot):
        pltpu.make_async_copy(
            w13_hbm.at[e, pl.ds(inter, inter)], u_buf.at[slot], sems.at[1, slot]
        ).start()

    def start_w2(e, slot):
        pltpu.make_async_copy(
            w2_hbm.at[e], w2_buf.at[slot], sems.at[2, slot]
        ).start()

    def wait(stream, slot, buf):
        pltpu.make_async_copy(
            w2_hbm.at[0], buf.at[slot], sems.at[stream, slot]
        ).wait()

    start_g(0, 0)
    start_u(0, 0)
    start_w2(0, 0)
    start_g(1, 1)
    start_u(1, 1)
    start_w2(1, 1)

    pltpu.make_async_copy(x_hbm, x_ref, io_sem).start()
    pltpu.make_async_copy(logits_hbm, logits_ref, io_sem).start()
    pltpu.make_async_copy(x_hbm, x_ref, io_sem).wait()
    pltpu.make_async_copy(logits_hbm, logits_ref, io_sem).wait()

    logits = logits_ref[...]
    m1 = jnp.max(logits, axis=-1, keepdims=True)
    idx1 = jnp.argmax(logits, axis=-1, keepdims=True)
    neg = jnp.finfo(jnp.float32).min
    cols = jax.lax.broadcasted_iota(jnp.int32, logits.shape, 1)
    masked = jnp.where(cols == idx1, neg, logits)
    m2 = jnp.max(masked, axis=-1, keepdims=True)
    idx2 = jnp.argmax(masked, axis=-1, keepdims=True)
    r = jnp.exp(m2 - m1)
    w1 = 1.0 / (1.0 + r)
    w2c = r / (1.0 + r)
    coeff = jnp.where(cols == idx1, w1, 0.0) + jnp.where(cols == idx2, w2c, 0.0)
    for ei in range(num_experts):
        coeff_ref[ei] = coeff[:, ei : ei + 1]
    out_ref[...] = jnp.zeros_like(out_ref)
    x = x_ref[...]

    def body(i, _):
        for s in range(2):
            e = 2 * i + s
            wait(0, s, g_buf)
            g = jax.lax.dot_general(
                x, g_buf[s], (((1,), (1,)), ((), ())),
                preferred_element_type=jnp.float32,
            )

            @pl.when(i < (num_experts // 2) - 1)
            def _pf_g():
                start_g(e + 2, s)

            wait(1, s, u_buf)
            u = jax.lax.dot_general(
                x, u_buf[s], (((1,), (1,)), ((), ())),
                preferred_element_type=jnp.float32,
            )

            @pl.when(i < (num_experts // 2) - 1)
            def _pf_u():
                start_u(e + 2, s)

            h = (g * jax.nn.sigmoid(g) * u) * coeff_ref[e]
            wait(2, s, w2_buf)
            y = jax.lax.dot_general(
                h, w2_buf[s], (((1,), (1,)), ((), ())),
                preferred_element_type=jnp.float32,
            )

            @pl.when(i < (num_experts // 2) - 1)
            def _pf_w2():
                start_w2(e + 2, s)

            out_ref[...] += y
        return 0

    jax.lax.fori_loop(0, num_experts // 2, body, 0)


def kernel(hidden_states, router_logits, w13_weight, w2_weight):
    tokens, hidden = hidden_states.shape
    num_experts = w13_weight.shape[0]
    inter = w2_weight.shape[2]
    return pl.pallas_call(
        _moe_kernel,
        in_specs=[
            pl.BlockSpec(memory_space=pl.ANY),
            pl.BlockSpec(memory_space=pl.ANY),
            pl.BlockSpec(memory_space=pl.ANY),
            pl.BlockSpec(memory_space=pl.ANY),
        ],
        out_specs=pl.BlockSpec((tokens, hidden), lambda: (0, 0)),
        out_shape=jax.ShapeDtypeStruct((tokens, hidden), jnp.float32),
        scratch_shapes=[
            pltpu.VMEM((tokens, hidden), jnp.float32),
            pltpu.VMEM((tokens, num_experts), jnp.float32),
            pltpu.VMEM((num_experts, tokens, 1), jnp.float32),
            pltpu.VMEM((2, inter, hidden), jnp.float32),
            pltpu.VMEM((2, inter, hidden), jnp.float32),
            pltpu.VMEM((2, hidden, inter), jnp.float32),
            pltpu.SemaphoreType.DMA((3, 2)),
            pltpu.SemaphoreType.DMA,
        ],
    )(hidden_states, router_logits, w13_weight, w2_weight)


# speedup vs baseline: 1.0139x; 1.0139x over previous
import jax
import jax.numpy as jnp
from jax.experimental import pallas as pl
from jax.experimental.pallas import tpu as pltpu


def _moe_kernel(
    x_hbm, logits_hbm, w13_hbm, w2_hbm, out_ref,
    x_ref, logits_ref, g_buf, u_buf, w2_buf, sems, io_sem
):
    num_experts, inter, hidden = w2_hbm.shape[0], w2_hbm.shape[2], w2_hbm.shape[1]

    def start_g(e, slot):
        pltpu.make_async_copy(
            w13_hbm.at[e, pl.ds(0, inter)], g_buf.at[slot], sems.at[0, slot]
        ).start()

    def start_u(e, slot):
        pltpu.make_async_copy(
            w13_hbm.at[e, pl.ds(inter, inter)], u_buf.at[slot], sems.at[1, slot]
        ).start()

    def start_w2(e, slot):
        pltpu.make_async_copy(
            w2_hbm.at[e], w2_buf.at[slot], sems.at[2, slot]
        ).start()

    def wait(stream, slot, buf):
        pltpu.make_async_copy(
            w2_hbm.at[0], buf.at[slot], sems.at[stream, slot]
        ).wait()

    pltpu.make_async_copy(x_hbm, x_ref, io_sem).start()
    pltpu.make_async_copy(logits_hbm, logits_ref, io_sem).start()
    start_g(0, 0)
    start_u(0, 0)
    start_w2(0, 0)
    start_g(1, 1)
    start_u(1, 1)
    start_w2(1, 1)
    pltpu.make_async_copy(x_hbm, x_ref, io_sem).wait()
    pltpu.make_async_copy(logits_hbm, logits_ref, io_sem).wait()

    logits = logits_ref[...]
    m1 = jnp.max(logits, axis=-1, keepdims=True)
    idx1 = jnp.argmax(logits, axis=-1, keepdims=True)
    neg = jnp.finfo(jnp.float32).min
    cols = jax.lax.broadcasted_iota(jnp.int32, logits.shape, 1)
    masked = jnp.where(cols == idx1, neg, logits)
    m2 = jnp.max(masked, axis=-1, keepdims=True)
    idx2 = jnp.argmax(masked, axis=-1, keepdims=True)
    r = jnp.exp(m2 - m1)
    w1 = 1.0 / (1.0 + r)
    w2c = r / (1.0 + r)
    coeff = jnp.where(cols == idx1, w1, 0.0) + jnp.where(cols == idx2, w2c, 0.0)
    out_ref[...] = jnp.zeros_like(out_ref)
    x = x_ref[...]

    for e in range(num_experts):
        s = e % 3
        ps = (e + 2) % 3
        wait(0, s, g_buf)
        g = jax.lax.dot_general(
            x, g_buf[s], (((1,), (1,)), ((), ())),
            preferred_element_type=jnp.float32,
        )
        if e + 2 < num_experts:
            start_g(e + 2, ps)
        wait(1, s, u_buf)
        u = jax.lax.dot_general(
            x, u_buf[s], (((1,), (1,)), ((), ())),
            preferred_element_type=jnp.float32,
        )
        if e + 2 < num_experts:
            start_u(e + 2, ps)
        h = (g * jax.nn.sigmoid(g) * u) * coeff[:, e : e + 1]
        wait(2, s, w2_buf)
        y = jax.lax.dot_general(
            h, w2_buf[s], (((1,), (1,)), ((), ())),
            preferred_element_type=jnp.float32,
        )
        if e + 2 < num_experts:
            start_w2(e + 2, ps)
        out_ref[...] += y


def kernel(hidden_states, router_logits, w13_weight, w2_weight):
    tokens, hidden = hidden_states.shape
    num_experts = w13_weight.shape[0]
    inter = w2_weight.shape[2]
    return pl.pallas_call(
        _moe_kernel,
        in_specs=[
            pl.BlockSpec(memory_space=pl.ANY),
            pl.BlockSpec(memory_space=pl.ANY),
            pl.BlockSpec(memory_space=pl.ANY),
            pl.BlockSpec(memory_space=pl.ANY),
        ],
        out_specs=pl.BlockSpec((tokens, hidden), lambda: (0, 0)),
        out_shape=jax.ShapeDtypeStruct((tokens, hidden), jnp.float32),
        scratch_shapes=[
            pltpu.VMEM((tokens, hidden), jnp.float32),
            pltpu.VMEM((tokens, num_experts), jnp.float32),
            pltpu.VMEM((3, inter, hidden), jnp.float32),
            pltpu.VMEM((3, inter, hidden), jnp.float32),
            pltpu.VMEM((3, hidden, inter), jnp.float32),
            pltpu.SemaphoreType.DMA((3, 3)),
            pltpu.SemaphoreType.DMA,
        ],
    )(hidden_states, router_logits, w13_weight, w2_weight)
